# baseline (speedup 1.0000x reference)
import functools

import jax
import jax.numpy as jnp
from jax import lax
from jax.experimental import pallas as pl
from jax.experimental.pallas import tpu as pltpu
from jax.experimental.pallas import tpu_sc as plsc

NUM_CORES = 2
NUM_SUBCORES = 16
NUM_WORKERS = NUM_CORES * NUM_SUBCORES
CHUNK = 8
NBUF = 11
LOOKAHEAD = 9
N_PHASE = 2


@jax.jit
def _embed_lookup(ids_flat, token_embedding, pos_flat):
    n_rows = ids_flat.shape[0]
    seq_len, embed = pos_flat.shape
    n_batch = n_rows // seq_len
    s_per_w = seq_len // NUM_WORKERS
    s_per_p = s_per_w // N_PHASE
    chunks_per_win = s_per_p // CHUNK
    n_chunks = N_PHASE * n_batch * chunks_per_win
    groups_per_row = embed // 16

    mesh = plsc.VectorSubcoreMesh(
        core_axis_name="c", subcore_axis_name="s",
        num_cores=NUM_CORES, num_subcores=NUM_SUBCORES,
    )

    @functools.partial(
        pl.kernel,
        out_type=jax.ShapeDtypeStruct((n_rows, embed), jnp.float32),
        mesh=mesh,
        scratch_types=(
            [pltpu.VMEM((n_batch * s_per_w,), jnp.int32),
             pltpu.VMEM((s_per_p, embed), jnp.float32)]
            + [pltpu.VMEM((CHUNK, embed), jnp.float32)] * NBUF
            + [pltpu.SemaphoreType.DMA] * (2 * NBUF + 2)
        ),
    )
    def emb_kernel(ids_hbm, tab_hbm, pos_hbm, out_hbm, idx_v, pos_v, *rest):
        nbuf = NBUF
        bufs = rest[:nbuf]
        gsems = rest[nbuf:2 * nbuf]
        osems = rest[2 * nbuf:3 * nbuf]
        isem, psem = rest[3 * nbuf], rest[3 * nbuf + 1]
        wid = lax.axis_index("s") * NUM_CORES + lax.axis_index("c")
        s0 = wid * s_per_w

        def load_pos(p):
            return pltpu.async_copy(
                pos_hbm.at[pl.ds(s0 + p * s_per_p, s_per_p)], pos_v, psem)

        pos_d = load_pos(0)
        id_d = [
            pltpu.async_copy(
                ids_hbm.at[pl.ds(b * seq_len + s0, s_per_w)],
                idx_v.at[pl.ds(b * s_per_w, s_per_w)],
                isem,
            )
            for b in range(n_batch)
        ]
        for d in id_d:
            d.wait()

        def chunk_coords(k):
            p, rest_k = divmod(k, n_batch * chunks_per_win)
            b, c = divmod(rest_k, chunks_per_win)
            win = p * s_per_p + c * CHUNK
            return b * s_per_w + win, b * seq_len + s0 + win, c * CHUNK

        def start_gather(k):
            i0, _, _ = chunk_coords(k)
            return pltpu.async_copy(
                tab_hbm.at[idx_v.at[pl.ds(i0, CHUNK)]],
                bufs[k % nbuf], gsems[k % nbuf],
            )

        gd = [None] * n_chunks
        od = [None] * n_chunks
        for k in range(min(LOOKAHEAD, n_chunks)):
            gd[k] = start_gather(k)
        per_phase = n_chunks // N_PHASE
        for k in range(n_chunks):
            buf = bufs[k % nbuf]
            _, r0, off = chunk_coords(k)
            if k + LOOKAHEAD < n_chunks:
                j = k + LOOKAHEAD - nbuf
                if j >= 0:
                    od[j].wait()
                gd[k + LOOKAHEAD] = start_gather(k + LOOKAHEAD)
            gd[k].wait()
            if k % per_phase == 0:
                pos_d.wait()

            @plsc.parallel_loop(0, groups_per_row, unroll=2)
            def add_body(i):
                g = i * 16
                for r in range(CHUNK):
                    plsc.addupdate(
                        buf.at[r, pl.ds(g, 16)],
                        pos_v[off + r, pl.ds(g, 16)],
                    )

            od[k] = pltpu.async_copy(buf, out_hbm.at[pl.ds(r0, CHUNK)],
                                     osems[k % nbuf])
            if k == per_phase - 1 and N_PHASE > 1:
                pos_d = load_pos(1)
        for k in range(max(0, n_chunks - nbuf), n_chunks):
            od[k].wait()

    return emb_kernel(ids_flat, token_embedding, pos_flat)


def kernel(input_ids, token_embedding, position_embeds):
    b, s = input_ids.shape
    embed = token_embedding.shape[1]
    ids_flat = input_ids.astype(jnp.int32).reshape(b * s)
    pos_flat = position_embeds[0, :s, :]
    out = _embed_lookup(ids_flat, token_embedding, pos_flat)
    return out.reshape(b, s, embed)


# speedup vs baseline: 1.0095x; 1.0095x over previous
import functools

import jax
import jax.numpy as jnp
from jax import lax
from jax.experimental import pallas as pl
from jax.experimental.pallas import tpu as pltpu
from jax.experimental.pallas import tpu_sc as plsc

NUM_CORES = 2
NUM_SUBCORES = 16
NUM_WORKERS = NUM_CORES * NUM_SUBCORES
CHUNK = 8
NBUF = 7
LOOKAHEAD = 5


@jax.jit
def _embed_lookup(ids_flat, token_embedding, pos_flat):
    n_rows = ids_flat.shape[0]
    seq_len, embed = pos_flat.shape
    n_batch = n_rows // seq_len
    s_per_w = seq_len // NUM_WORKERS
    chunks_per_batch = s_per_w // CHUNK
    n_chunks = n_batch * chunks_per_batch
    groups_per_row = embed // 16

    mesh = plsc.VectorSubcoreMesh(
        core_axis_name="c", subcore_axis_name="s",
        num_cores=NUM_CORES, num_subcores=NUM_SUBCORES,
    )

    @functools.partial(
        pl.kernel,
        out_type=jax.ShapeDtypeStruct((n_rows, embed), jnp.float32),
        mesh=mesh,
        scratch_types=(
            [pltpu.VMEM((n_batch * s_per_w,), jnp.int32),
             pltpu.VMEM((s_per_w, embed), jnp.float32)]
            + [pltpu.VMEM((CHUNK, embed), jnp.float32)] * NBUF
            + [pltpu.SemaphoreType.DMA] * (2 * NBUF + 2)
        ),
    )
    def emb_kernel(ids_hbm, tab_hbm, pos_hbm, out_hbm, idx_v, pos_v, *rest):
        nbuf = NBUF
        bufs = rest[:nbuf]
        gsems = rest[nbuf:2 * nbuf]
        osems = rest[2 * nbuf:3 * nbuf]
        isem, psem = rest[3 * nbuf], rest[3 * nbuf + 1]
        wid = lax.axis_index("s") * NUM_CORES + lax.axis_index("c")
        s0 = wid * s_per_w
        pos_d = pltpu.async_copy(pos_hbm.at[pl.ds(s0, s_per_w)], pos_v, psem)
        id_d = [
            pltpu.async_copy(
                ids_hbm.at[pl.ds(b * seq_len + s0, s_per_w)],
                idx_v.at[pl.ds(b * s_per_w, s_per_w)],
                isem,
            )
            for b in range(n_batch)
        ]
        for d in id_d:
            d.wait()

        def chunk_coords(k):
            b, c = divmod(k, chunks_per_batch)
            off = c * CHUNK
            return b * s_per_w + off, b * seq_len + s0 + off, off

        def start_gather(k):
            i0, _, _ = chunk_coords(k)
            return pltpu.async_copy(
                tab_hbm.at[idx_v.at[pl.ds(i0, CHUNK)]],
                bufs[k % nbuf], gsems[k % nbuf],
            )

        gd = [None] * n_chunks
        od = [None] * n_chunks
        for k in range(min(LOOKAHEAD, n_chunks)):
            gd[k] = start_gather(k)
        for k in range(n_chunks):
            buf = bufs[k % nbuf]
            _, r0, off = chunk_coords(k)
            if k + LOOKAHEAD < n_chunks:
                j = k + LOOKAHEAD - nbuf
                if j >= 0:
                    od[j].wait()
                gd[k + LOOKAHEAD] = start_gather(k + LOOKAHEAD)
            gd[k].wait()
            if k == 0:
                pos_d.wait()

            @plsc.parallel_loop(0, groups_per_row, unroll=4)
            def add_body(i):
                g = i * 16
                for r in range(CHUNK):
                    plsc.addupdate(
                        buf.at[r, pl.ds(g, 16)],
                        pos_v[off + r, pl.ds(g, 16)],
                    )

            od[k] = pltpu.async_copy(buf, out_hbm.at[pl.ds(r0, CHUNK)],
                                     osems[k % nbuf])
        for k in range(max(0, n_chunks - nbuf), n_chunks):
            od[k].wait()

    return emb_kernel(ids_flat, token_embedding, pos_flat)


def kernel(input_ids, token_embedding, position_embeds):
    b, s = input_ids.shape
    embed = token_embedding.shape[1]
    ids_flat = input_ids.astype(jnp.int32).reshape(b * s)
    pos_flat = position_embeds[0, :s, :]
    out = _embed_lookup(ids_flat, token_embedding, pos_flat)
    return out.reshape(b, s, embed)


# speedup vs baseline: 1.0262x; 1.0166x over previous
import functools

import jax
import jax.numpy as jnp
from jax import lax
from jax.experimental import pallas as pl
from jax.experimental.pallas import tpu as pltpu
from jax.experimental.pallas import tpu_sc as plsc

NUM_CORES = 2
NUM_SUBCORES = 16
NUM_WORKERS = NUM_CORES * NUM_SUBCORES
CHUNK = 8
NBUF = 7
LOOKAHEAD = 5


@jax.jit
def _embed_lookup(ids_flat, token_embedding, pos_flat):
    n_rows = ids_flat.shape[0]
    seq_len, embed = pos_flat.shape
    n_batch = n_rows // seq_len
    s_per_w = seq_len // NUM_WORKERS
    chunks_per_batch = s_per_w // CHUNK
    n_chunks = n_batch * chunks_per_batch
    groups_per_row = embed // 16

    mesh = plsc.VectorSubcoreMesh(
        core_axis_name="c", subcore_axis_name="s",
        num_cores=NUM_CORES, num_subcores=NUM_SUBCORES,
    )

    @functools.partial(
        pl.kernel,
        out_type=jax.ShapeDtypeStruct((n_rows, embed), jnp.float32),
        mesh=mesh,
        scratch_types=(
            [pltpu.VMEM((n_batch * s_per_w,), jnp.int32),
             pltpu.VMEM((s_per_w, embed), jnp.float32)]
            + [pltpu.VMEM((CHUNK, embed), jnp.float32)] * NBUF
            + [pltpu.SemaphoreType.DMA] * (2 * NBUF + 2)
        ),
    )
    def emb_kernel(ids_hbm, tab_hbm, pos_hbm, out_hbm, idx_v, pos_v, *rest):
        nbuf = NBUF
        bufs = rest[:nbuf]
        gsems = rest[nbuf:2 * nbuf]
        osems = rest[2 * nbuf:3 * nbuf]
        isem, psem = rest[3 * nbuf], rest[3 * nbuf + 1]
        wid = lax.axis_index("s") * NUM_CORES + lax.axis_index("c")
        s0 = wid * s_per_w
        pos_d = pltpu.async_copy(pos_hbm.at[pl.ds(s0, s_per_w)], pos_v, psem)
        id_d = [
            pltpu.async_copy(
                ids_hbm.at[pl.ds(b * seq_len + s0, s_per_w)],
                idx_v.at[pl.ds(b * s_per_w, s_per_w)],
                isem,
            )
            for b in range(n_batch)
        ]
        for d in id_d:
            d.wait()

        def chunk_coords(k):
            b, c = divmod(k, chunks_per_batch)
            off = c * CHUNK
            return b * s_per_w + off, b * seq_len + s0 + off, off

        def start_gather(k):
            i0, _, _ = chunk_coords(k)
            return pltpu.async_copy(
                tab_hbm.at[idx_v.at[pl.ds(i0, CHUNK)]],
                bufs[k % nbuf], gsems[k % nbuf],
            )

        gd = [None] * n_chunks
        od = [None] * n_chunks
        for k in range(min(LOOKAHEAD, n_chunks)):
            gd[k] = start_gather(k)
        for k in range(n_chunks):
            buf = bufs[k % nbuf]
            _, r0, off = chunk_coords(k)
            if k + LOOKAHEAD < n_chunks:
                j = k + LOOKAHEAD - nbuf
                if j >= 0:
                    od[j].wait()
                gd[k + LOOKAHEAD] = start_gather(k + LOOKAHEAD)
            gd[k].wait()
            if k == 0:
                pos_d.wait()

            @plsc.parallel_loop(0, groups_per_row, unroll=2)
            def add_body(i):
                g = i * 16
                for r in range(CHUNK):
                    plsc.addupdate(
                        buf.at[r, pl.ds(g, 16)],
                        pos_v[off + r, pl.ds(g, 16)],
                    )

            od[k] = pltpu.async_copy(buf, out_hbm.at[pl.ds(r0, CHUNK)],
                                     osems[k % nbuf])
        for k in range(max(0, n_chunks - nbuf), n_chunks):
            od[k].wait()

    return emb_kernel(ids_flat, token_embedding, pos_flat)


def kernel(input_ids, token_embedding, position_embeds):
    b, s = input_ids.shape
    embed = token_embedding.shape[1]
    ids_flat = input_ids.astype(jnp.int32).reshape(b * s)
    pos_flat = position_embeds[0, :s, :]
    out = _embed_lookup(ids_flat, token_embedding, pos_flat)
    return out.reshape(b, s, embed)
